# Initial kernel scaffold; baseline (speedup 1.0000x reference)
import jax
import jax.numpy as jnp
from jax.experimental import pallas as pl


def kernel(feats, coords, W, gamma, beta):
    raise NotImplementedError("write your pallas kernel here")


# speedup vs baseline: 7.4436x; 7.4436x over previous
import functools

import jax
import jax.numpy as jnp
from jax import lax
from jax.experimental import pallas as pl
from jax.experimental.pallas import tpu as pltpu
from jax.experimental.pallas import tpu_sc as plsc

GRID = 128
G = GRID + 2
G3 = G * G * G
C = 32
NTAP = 27

NW = 32
TILE = 128
LANES = 16


def _mm_body(f_ref, w_ref, y_ref):
    f = f_ref[...]
    for k in range(NTAP):
        y_ref[k] = jnp.dot(f, w_ref[k], preferred_element_type=jnp.float32)


def _tap_matmuls(feats_pad, W, npad, bl):
    return pl.pallas_call(
        _mm_body,
        grid=(npad // bl,),
        in_specs=[
            pl.BlockSpec((bl, C), lambda i: (i, 0)),
            pl.BlockSpec((NTAP, C, C), lambda i: (0, 0, 0)),
        ],
        out_specs=pl.BlockSpec((NTAP, bl, C), lambda i: (0, i, 0)),
        out_shape=jax.ShapeDtypeStruct((NTAP, npad, C), jnp.float32),
    )(feats_pad, W)


def _make_sc_gather(npad):
    rows_per_w = npad // NW
    ntiles = rows_per_w // TILE
    mesh = plsc.VectorSubcoreMesh(core_axis_name="c", subcore_axis_name="s")

    @functools.partial(
        pl.kernel,
        mesh=mesh,
        compiler_params=pltpu.CompilerParams(use_tc_tiling_on_sc=False),
        out_type=jax.ShapeDtypeStruct((npad, C), jnp.float32),
        scratch_types=[
            pltpu.VMEM((TILE,), jnp.int32),
            pltpu.VMEM((TILE,), jnp.int32),
            pltpu.VMEM((TILE,), jnp.int32),
            pltpu.VMEM((TILE, C), jnp.float32),
            pltpu.VMEM((TILE, C), jnp.float32),
            pltpu.SemaphoreType.DMA,
        ],
    )
    def sc_gather(y_hbm, table_hbm, h_hbm, out_hbm, h_v, q_v, idx_v, rows_v,
                  acc_v, sem):
        wid = lax.axis_index("s") * 2 + lax.axis_index("c")
        base = wid * rows_per_w

        def tile_body(t, _):
            row0 = base + t * TILE
            pltpu.sync_copy(h_hbm.at[pl.ds(row0, TILE)], h_v)
            for i in range(TILE):
                for j in range(C // LANES):
                    acc_v[i, pl.ds(j * LANES, LANES)] = jnp.zeros(
                        (LANES,), jnp.float32)

            def tap_body(k, _):
                dx = k // 9 - 1
                dy = (k // 3) % 3 - 1
                dz = k % 3 - 1
                off = (dx * G + dy) * G + dz
                for j in range(TILE // LANES):
                    hv = h_v[pl.ds(j * LANES, LANES)]
                    q_v[pl.ds(j * LANES, LANES)] = jnp.where(
                        hv < 0, 0, hv + off)
                pltpu.async_copy(table_hbm.at[q_v], idx_v, sem).wait()
                koff = k * npad
                for j in range(TILE // LANES):
                    sl = pl.ds(j * LANES, LANES)
                    idx_v[sl] = idx_v[sl] + koff
                pltpu.async_copy(y_hbm.at[idx_v], rows_v, sem).wait()
                for i in range(TILE):
                    for j in range(C // LANES):
                        sl = pl.ds(j * LANES, LANES)
                        plsc.addupdate(acc_v.at[i, sl], rows_v[i, sl])
                return 0

            lax.fori_loop(0, NTAP, tap_body, 0)
            pltpu.sync_copy(acc_v, out_hbm.at[pl.ds(row0, TILE)])
            return 0

        lax.fori_loop(0, ntiles, tile_body, 0)

    return sc_gather


def _bn_body(n_true, x_ref, g_ref, b_ref, o_ref, s_ref, q_ref):
    p = pl.program_id(0)
    i = pl.program_id(1)

    @pl.when((p == 0) & (i == 0))
    def _():
        s_ref[...] = jnp.zeros_like(s_ref)
        q_ref[...] = jnp.zeros_like(q_ref)

    @pl.when(p == 0)
    def _():
        x = x_ref[...]
        s_ref[0:1, :] += jnp.sum(x, axis=0, keepdims=True)
        q_ref[0:1, :] += jnp.sum(x * x, axis=0, keepdims=True)
        o_ref[...] = jnp.zeros_like(x)

    @pl.when(p == 1)
    def _():
        x = x_ref[...]
        inv_n = 1.0 / float(n_true)
        mean = s_ref[0:1, :] * inv_n
        var = q_ref[0:1, :] * inv_n - mean * mean
        inv = lax.rsqrt(var + 1e-5)
        g = g_ref[0:1, :]
        b = b_ref[0:1, :]
        o_ref[...] = jnp.maximum((x - mean) * inv * g + b, 0.0)


def _bn_relu(out_pad, gamma, beta, n_true, npad, bl):
    g8 = jnp.broadcast_to(gamma[None, :], (8, C))
    b8 = jnp.broadcast_to(beta[None, :], (8, C))
    return pl.pallas_call(
        functools.partial(_bn_body, n_true),
        grid=(2, npad // bl),
        in_specs=[
            pl.BlockSpec((bl, C), lambda p, i: (i, 0)),
            pl.BlockSpec((8, C), lambda p, i: (0, 0)),
            pl.BlockSpec((8, C), lambda p, i: (0, 0)),
        ],
        out_specs=pl.BlockSpec((bl, C), lambda p, i: (i, 0)),
        out_shape=jax.ShapeDtypeStruct((npad, C), jnp.float32),
        scratch_shapes=[
            pltpu.VMEM((8, C), jnp.float32),
            pltpu.VMEM((8, C), jnp.float32),
        ],
    )(out_pad, g8, b8)


def kernel(feats, coords, W, gamma, beta):
    n = feats.shape[0]
    chunk = NW * TILE
    npad = ((n + 1 + chunk - 1) // chunk) * chunk

    h = ((coords[:, 0] + 1) * G + (coords[:, 1] + 1)) * G + (coords[:, 2] + 1)
    table = jnp.full((G3,), n, jnp.int32).at[h].min(
        jnp.arange(n, dtype=jnp.int32))
    feats_pad = jnp.zeros((npad, C), jnp.float32).at[:n, :].set(feats)
    h_pad = jnp.full((npad,), -1, jnp.int32).at[:n].set(h)

    y = _tap_matmuls(feats_pad, W, npad, 512)
    y2 = y.reshape(NTAP * npad, C)
    out_pad = _make_sc_gather(npad)(y2, table, h_pad)
    out = _bn_relu(out_pad, gamma, beta, n, npad, 2048)
    return out[:n]


# speedup vs baseline: 10.1075x; 1.3579x over previous
import functools

import jax
import jax.numpy as jnp
from jax import lax
from jax.experimental import pallas as pl
from jax.experimental.pallas import tpu as pltpu
from jax.experimental.pallas import tpu_sc as plsc

GRID = 128
G = GRID + 2
G3 = G * G * G
C = 32
NTAP = 27

NW = 32
TILE = 128
LANES = 16


def _mm_body(f_ref, w_ref, y_ref):
    f = f_ref[...]
    for k in range(NTAP):
        y_ref[k] = jnp.dot(f, w_ref[k], preferred_element_type=jnp.float32)


def _tap_matmuls(feats_pad, W, npad, bl):
    return pl.pallas_call(
        _mm_body,
        grid=(npad // bl,),
        in_specs=[
            pl.BlockSpec((bl, C), lambda i: (i, 0)),
            pl.BlockSpec((NTAP, C, C), lambda i: (0, 0, 0)),
        ],
        out_specs=pl.BlockSpec((NTAP, bl, C), lambda i: (0, i, 0)),
        out_shape=jax.ShapeDtypeStruct((NTAP, npad, C), jnp.float32),
    )(feats_pad, W)


def _make_sc_gather(npad):
    rows_per_w = npad // NW
    ntiles = rows_per_w // TILE
    mesh = plsc.VectorSubcoreMesh(core_axis_name="c", subcore_axis_name="s")

    ngrp = 3
    gsz = NTAP // ngrp

    @functools.partial(
        pl.kernel,
        mesh=mesh,
        compiler_params=pltpu.CompilerParams(use_tc_tiling_on_sc=False),
        out_type=jax.ShapeDtypeStruct((npad, C), jnp.float32),
        scratch_types=[
            pltpu.VMEM((TILE,), jnp.int32),
            pltpu.VMEM((NTAP, TILE), jnp.int32),
            pltpu.VMEM((NTAP, TILE), jnp.int32),
            pltpu.VMEM((gsz, TILE, C), jnp.float32),
            pltpu.VMEM((gsz, TILE, C), jnp.float32),
            pltpu.VMEM((TILE, C), jnp.float32),
            pltpu.SemaphoreType.DMA,
            pltpu.SemaphoreType.DMA,
            pltpu.SemaphoreType.DMA,
        ],
    )
    def sc_gather(y_hbm, table_hbm, h_hbm, out_hbm, h_v, q_v, idx_v, rows_a,
                  rows_b, acc_v, sem_i, sem_a, sem_b):
        wid = lax.axis_index("s") * 2 + lax.axis_index("c")
        base = wid * rows_per_w
        rows_bufs = (rows_a, rows_b)
        row_sems = (sem_a, sem_b)

        def accum_group(buf):
            def body(kk, _):
                for i in range(TILE):
                    for j in range(C // LANES):
                        sl = pl.ds(j * LANES, LANES)
                        plsc.addupdate(acc_v.at[i, sl], buf[kk, i, sl])
                return 0
            lax.fori_loop(0, gsz, body, 0)

        def tile_body(t, _):
            row0 = base + t * TILE
            pltpu.sync_copy(h_hbm.at[pl.ds(row0, TILE)], h_v)
            for i in range(TILE):
                for j in range(C // LANES):
                    acc_v[i, pl.ds(j * LANES, LANES)] = jnp.zeros(
                        (LANES,), jnp.float32)

            idx_copies = []
            for k in range(NTAP):
                dx, dy, dz = k // 9 - 1, (k // 3) % 3 - 1, k % 3 - 1
                off = (dx * G + dy) * G + dz
                for j in range(TILE // LANES):
                    sl = pl.ds(j * LANES, LANES)
                    hv = h_v[sl]
                    q_v[k, sl] = jnp.where(hv < 0, 0, hv + off)
                idx_copies.append(
                    pltpu.async_copy(table_hbm.at[q_v.at[k]], idx_v.at[k],
                                     sem_i))

            row_copies = [None] * ngrp
            for g in range(ngrp):
                buf = rows_bufs[g % 2]
                sem = row_sems[g % 2]
                grp = []
                for kk in range(gsz):
                    k = g * gsz + kk
                    idx_copies[k].wait()
                    for j in range(TILE // LANES):
                        sl = pl.ds(j * LANES, LANES)
                        idx_v[k, sl] = idx_v[k, sl] + (k * npad)
                    grp.append(
                        pltpu.async_copy(y_hbm.at[idx_v.at[k]], buf.at[kk],
                                         sem))
                row_copies[g] = grp
                if g > 0:
                    for c in row_copies[g - 1]:
                        c.wait()
                    accum_group(rows_bufs[(g - 1) % 2])
            for c in row_copies[ngrp - 1]:
                c.wait()
            accum_group(rows_bufs[(ngrp - 1) % 2])

            pltpu.sync_copy(acc_v, out_hbm.at[pl.ds(row0, TILE)])
            return 0

        lax.fori_loop(0, ntiles, tile_body, 0)

    return sc_gather


def _bn_body(n_true, x_ref, g_ref, b_ref, o_ref, s_ref, q_ref):
    p = pl.program_id(0)
    i = pl.program_id(1)

    @pl.when((p == 0) & (i == 0))
    def _():
        s_ref[...] = jnp.zeros_like(s_ref)
        q_ref[...] = jnp.zeros_like(q_ref)

    @pl.when(p == 0)
    def _():
        x = x_ref[...]
        s_ref[0:1, :] += jnp.sum(x, axis=0, keepdims=True)
        q_ref[0:1, :] += jnp.sum(x * x, axis=0, keepdims=True)
        o_ref[...] = jnp.zeros_like(x)

    @pl.when(p == 1)
    def _():
        x = x_ref[...]
        inv_n = 1.0 / float(n_true)
        mean = s_ref[0:1, :] * inv_n
        var = q_ref[0:1, :] * inv_n - mean * mean
        inv = lax.rsqrt(var + 1e-5)
        g = g_ref[0:1, :]
        b = b_ref[0:1, :]
        o_ref[...] = jnp.maximum((x - mean) * inv * g + b, 0.0)


def _bn_relu(out_pad, gamma, beta, n_true, npad, bl):
    g8 = jnp.broadcast_to(gamma[None, :], (8, C))
    b8 = jnp.broadcast_to(beta[None, :], (8, C))
    return pl.pallas_call(
        functools.partial(_bn_body, n_true),
        grid=(2, npad // bl),
        in_specs=[
            pl.BlockSpec((bl, C), lambda p, i: (i, 0)),
            pl.BlockSpec((8, C), lambda p, i: (0, 0)),
            pl.BlockSpec((8, C), lambda p, i: (0, 0)),
        ],
        out_specs=pl.BlockSpec((bl, C), lambda p, i: (i, 0)),
        out_shape=jax.ShapeDtypeStruct((npad, C), jnp.float32),
        scratch_shapes=[
            pltpu.VMEM((8, C), jnp.float32),
            pltpu.VMEM((8, C), jnp.float32),
        ],
    )(out_pad, g8, b8)


def kernel(feats, coords, W, gamma, beta):
    n = feats.shape[0]
    chunk = NW * TILE
    npad = ((n + 1 + chunk - 1) // chunk) * chunk

    h = ((coords[:, 0] + 1) * G + (coords[:, 1] + 1)) * G + (coords[:, 2] + 1)
    table = jnp.full((G3,), n, jnp.int32).at[h].min(
        jnp.arange(n, dtype=jnp.int32))
    feats_pad = jnp.zeros((npad, C), jnp.float32).at[:n, :].set(feats)
    h_pad = jnp.full((npad,), -1, jnp.int32).at[:n].set(h)

    y = _tap_matmuls(feats_pad, W, npad, 512)
    y2 = y.reshape(NTAP * npad, C)
    out_pad = _make_sc_gather(npad)(y2, table, h_pad)
    out = _bn_relu(out_pad, gamma, beta, n, npad, 2048)
    return out[:n]


# speedup vs baseline: 10.3046x; 1.0195x over previous
import functools

import jax
import jax.numpy as jnp
from jax import lax
from jax.experimental import pallas as pl
from jax.experimental.pallas import tpu as pltpu
from jax.experimental.pallas import tpu_sc as plsc

GRID = 128
G = GRID + 2
G3 = G * G * G
C = 32
NTAP = 27

NW = 32
TILE = 128
LANES = 16


def _mm_body(f_ref, w_ref, y_ref):
    f = f_ref[...]
    for k in range(NTAP):
        y_ref[k] = jnp.dot(f, w_ref[k], preferred_element_type=jnp.float32)


def _tap_matmuls(feats_pad, W, npad, bl):
    return pl.pallas_call(
        _mm_body,
        grid=(npad // bl,),
        in_specs=[
            pl.BlockSpec((bl, C), lambda i: (i, 0)),
            pl.BlockSpec((NTAP, C, C), lambda i: (0, 0, 0)),
        ],
        out_specs=pl.BlockSpec((NTAP, bl, C), lambda i: (0, i, 0)),
        out_shape=jax.ShapeDtypeStruct((NTAP, npad, C), jnp.float32),
    )(feats_pad, W)


def _make_sc_gather(npad):
    rows_per_w = npad // NW
    ntiles = rows_per_w // TILE
    mesh = plsc.VectorSubcoreMesh(core_axis_name="c", subcore_axis_name="s")

    ngrp = 3
    gsz = NTAP // ngrp

    @functools.partial(
        pl.kernel,
        mesh=mesh,
        compiler_params=pltpu.CompilerParams(use_tc_tiling_on_sc=False),
        out_type=jax.ShapeDtypeStruct((npad, C), jnp.float32),
        scratch_types=[
            pltpu.VMEM((TILE,), jnp.int32),
            pltpu.VMEM((NTAP, TILE), jnp.int32),
            pltpu.VMEM((NTAP, TILE), jnp.int32),
            pltpu.VMEM((NTAP, TILE, C), jnp.float32),
            pltpu.VMEM((TILE, C), jnp.float32),
            pltpu.SemaphoreType.DMA,
            pltpu.SemaphoreType.DMA,
        ],
    )
    def sc_gather(y_hbm, table_hbm, h_hbm, out_hbm, h_v, q_v, idx_v, rows_v,
                  acc_v, sem_i, sem_r):
        wid = lax.axis_index("s") * 2 + lax.axis_index("c")
        base = wid * rows_per_w

        def accum_group(g):
            def body(kk, _):
                for i in range(TILE):
                    for j in range(C // LANES):
                        sl = pl.ds(j * LANES, LANES)
                        plsc.addupdate(acc_v.at[i, sl],
                                       rows_v[g * gsz + kk, i, sl])
                return 0
            lax.fori_loop(0, gsz, body, 0)

        def tile_body(t, _):
            row0 = base + t * TILE
            pltpu.sync_copy(h_hbm.at[pl.ds(row0, TILE)], h_v)

            idx_copies = []
            for k in range(NTAP):
                dx, dy, dz = k // 9 - 1, (k // 3) % 3 - 1, k % 3 - 1
                off = (dx * G + dy) * G + dz
                for j in range(TILE // LANES):
                    sl = pl.ds(j * LANES, LANES)
                    hv = h_v[sl]
                    q_v[k, sl] = jnp.where(hv < 0, 0, hv + off)
                idx_copies.append(
                    pltpu.async_copy(table_hbm.at[q_v.at[k]], idx_v.at[k],
                                     sem_i))

            for i in range(TILE):
                for j in range(C // LANES):
                    acc_v[i, pl.ds(j * LANES, LANES)] = jnp.zeros(
                        (LANES,), jnp.float32)

            row_copies = []
            for k in range(NTAP):
                idx_copies[k].wait()
                for j in range(TILE // LANES):
                    sl = pl.ds(j * LANES, LANES)
                    idx_v[k, sl] = idx_v[k, sl] + (k * npad)
                row_copies.append(
                    pltpu.async_copy(y_hbm.at[idx_v.at[k]], rows_v.at[k],
                                     sem_r))

            for g in range(ngrp):
                for kk in range(gsz):
                    row_copies[g * gsz + kk].wait()
                accum_group(g)

            pltpu.sync_copy(acc_v, out_hbm.at[pl.ds(row0, TILE)])
            return 0

        lax.fori_loop(0, ntiles, tile_body, 0)

    return sc_gather


def _bn_body(n_true, x_ref, g_ref, b_ref, o_ref, s_ref, q_ref):
    p = pl.program_id(0)
    i = pl.program_id(1)

    @pl.when((p == 0) & (i == 0))
    def _():
        s_ref[...] = jnp.zeros_like(s_ref)
        q_ref[...] = jnp.zeros_like(q_ref)

    @pl.when(p == 0)
    def _():
        x = x_ref[...]
        s_ref[0:1, :] += jnp.sum(x, axis=0, keepdims=True)
        q_ref[0:1, :] += jnp.sum(x * x, axis=0, keepdims=True)
        o_ref[...] = jnp.zeros_like(x)

    @pl.when(p == 1)
    def _():
        x = x_ref[...]
        inv_n = 1.0 / float(n_true)
        mean = s_ref[0:1, :] * inv_n
        var = q_ref[0:1, :] * inv_n - mean * mean
        inv = lax.rsqrt(var + 1e-5)
        g = g_ref[0:1, :]
        b = b_ref[0:1, :]
        o_ref[...] = jnp.maximum((x - mean) * inv * g + b, 0.0)


def _bn_relu(out_pad, gamma, beta, n_true, npad, bl):
    g8 = jnp.broadcast_to(gamma[None, :], (8, C))
    b8 = jnp.broadcast_to(beta[None, :], (8, C))
    return pl.pallas_call(
        functools.partial(_bn_body, n_true),
        grid=(2, npad // bl),
        in_specs=[
            pl.BlockSpec((bl, C), lambda p, i: (i, 0)),
            pl.BlockSpec((8, C), lambda p, i: (0, 0)),
            pl.BlockSpec((8, C), lambda p, i: (0, 0)),
        ],
        out_specs=pl.BlockSpec((bl, C), lambda p, i: (i, 0)),
        out_shape=jax.ShapeDtypeStruct((npad, C), jnp.float32),
        scratch_shapes=[
            pltpu.VMEM((8, C), jnp.float32),
            pltpu.VMEM((8, C), jnp.float32),
        ],
    )(out_pad, g8, b8)


def kernel(feats, coords, W, gamma, beta):
    n = feats.shape[0]
    chunk = NW * TILE
    npad = ((n + 1 + chunk - 1) // chunk) * chunk

    h = ((coords[:, 0] + 1) * G + (coords[:, 1] + 1)) * G + (coords[:, 2] + 1)
    table = jnp.full((G3,), n, jnp.int32).at[h].min(
        jnp.arange(n, dtype=jnp.int32))
    feats_pad = jnp.zeros((npad, C), jnp.float32).at[:n, :].set(feats)
    h_pad = jnp.full((npad,), -1, jnp.int32).at[:n].set(h)

    y = _tap_matmuls(feats_pad, W, npad, 512)
    y2 = y.reshape(NTAP * npad, C)
    out_pad = _make_sc_gather(npad)(y2, table, h_pad)
    out = _bn_relu(out_pad, gamma, beta, n, npad, 2048)
    return out[:n]
